# Initial kernel scaffold; baseline (speedup 1.0000x reference)
import jax
import jax.numpy as jnp
from jax.experimental import pallas as pl


def kernel(y):
    raise NotImplementedError("write your pallas kernel here")


# speedup vs baseline: 8.7088x; 8.7088x over previous
import functools

import jax
import jax.numpy as jnp
import numpy as np
from jax import lax
from jax.experimental import pallas as pl
from jax.experimental.pallas import tpu as pltpu
from jax.experimental.pallas import tpu_sc as plsc

_R = 128
_N = 32768
_K = 256
_L = 16
_NC = 2
_NS = 16
_NW = _NC * _NS
_ROWS_PER_W = _R // _NW
_SLICES = _N // _L
_UNROLL = 16
_TOPBIT = np.int32(-(2**31))

_FILTER = np.float32(0.75)
_MAGNIFY = np.float32(1.25)


def _sort_key(x):
    b = plsc.bitcast(x, jnp.int32)
    return b ^ ((b >> 31) & np.int32(0x7FFFFFFF))


def _rank_body(y_hbm, out_hbm, row_v, key_v):
    cid = lax.axis_index("c")
    sid = lax.axis_index("s")
    wid = sid * _NC + cid

    def per_row(r, _):
        row = wid * _ROWS_PER_W + r
        pltpu.sync_copy(y_hbm.at[row], row_v)

        def key_pass(i, carry):
            for u in range(_UNROLL):
                off = (i * _UNROLL + u) * _L
                key_v[pl.ds(off, _L)] = _sort_key(row_v[pl.ds(off, _L)])
            return carry
        lax.fori_loop(0, _SLICES // _UNROLL, key_pass, np.int32(0))

        def bit_step(jj, p):
            c = p | jnp.left_shift(1, 31 - jj)
            cs = c ^ _TOPBIT

            def count_scan(i, acc):
                for u in range(_UNROLL):
                    off = (i * _UNROLL + u) * _L
                    ks = key_v[pl.ds(off, _L)]
                    acc = acc + (ks >= cs).astype(jnp.int32)
                return acc
            zero_v = lax.iota(jnp.int32, _L) * np.int32(0)
            acc = lax.fori_loop(0, _SLICES // _UNROLL, count_scan, zero_v)
            cnt = jnp.sum(acc)
            return jnp.where(cnt >= _K, c, p)

        p = lax.fori_loop(0, 32, bit_step, np.int32(0))
        t_key = p ^ _TOPBIT

        def scale_pass(i, carry):
            for u in range(_UNROLL):
                off = (i * _UNROLL + u) * _L
                x = row_v[pl.ds(off, _L)]
                ks = key_v[pl.ds(off, _L)]
                row_v[pl.ds(off, _L)] = jnp.where(
                    ks < t_key, x * _FILTER, x * _MAGNIFY)
            return carry
        lax.fori_loop(0, _SLICES // _UNROLL, scale_pass, np.int32(0))

        pltpu.sync_copy(row_v, out_hbm.at[row])
        return np.int32(0)

    lax.fori_loop(0, _ROWS_PER_W, per_row, np.int32(0))


_rank_sc = functools.partial(
    pl.kernel,
    out_type=jax.ShapeDtypeStruct((_R, _N), jnp.float32),
    mesh=plsc.VectorSubcoreMesh(core_axis_name="c", subcore_axis_name="s"),
    scratch_types=[
        pltpu.VMEM((_N,), jnp.float32),
        pltpu.VMEM((_N,), jnp.int32),
    ],
    compiler_params=pltpu.CompilerParams(needs_layout_passes=False),
)(_rank_body)


def kernel(y):
    return _rank_sc(y)
